# baseline (speedup 1.0000x reference)
import functools

import jax
import jax.numpy as jnp
from jax import lax
from jax.experimental import pallas as pl
from jax.experimental.pallas import tpu as pltpu
from jax.experimental.pallas import tpu_sc as plsc

NC = 2
NS = 16


def _sc_scatter_acc(idx, mask, zeros):
    B, OD = mask.shape
    NW = NC * NS
    rpt = B // NW
    rps = B // NS

    mesh = plsc.VectorSubcoreMesh(core_axis_name="c", subcore_axis_name="s")

    @functools.partial(
        pl.kernel,
        out_type=jax.ShapeDtypeStruct((NC, B, OD), jnp.float32),
        mesh=mesh,
        scratch_types=[
            pltpu.VMEM((rpt,), jnp.int32),
            pltpu.VMEM((rpt, OD), jnp.float32),
            pltpu.VMEM_SHARED((B, OD), jnp.float32),
        ],
    )
    def sc_scatter(idx_hbm, mask_hbm, zero_hbm, acc_hbm, idx_v, mask_v, acc_sh):
        c = lax.axis_index("c")
        s = lax.axis_index("s")
        tid = c * NS + s
        pltpu.sync_copy(zero_hbm.at[pl.ds(s * rps, rps)],
                        acc_sh.at[pl.ds(s * rps, rps)])
        pltpu.sync_copy(idx_hbm.at[pl.ds(tid * rpt, rpt)], idx_v)
        pltpu.sync_copy(mask_hbm.at[pl.ds(tid * rpt, rpt)], mask_v)
        plsc.subcore_barrier()
        pltpu.sync_copy(mask_v, acc_sh.at[idx_v], add=True)
        plsc.subcore_barrier()
        pltpu.sync_copy(acc_sh.at[pl.ds(s * rps, rps)],
                        acc_hbm.at[c, pl.ds(s * rps, rps)])

    return sc_scatter(idx, mask, zeros)


def _tc_reduce_add(X, weight, acc, block_b):
    B, OD, IN = X.shape

    def body(w_ref, x_ref, acc_ref, o_ref):
        x = x_ref[...]
        w = w_ref[...]
        wb = jnp.broadcast_to(w[None], (x.shape[0], 1, IN))
        red = jax.lax.dot_general(
            wb, x,
            dimension_numbers=(((2,), (2,)), ((0,), (0,))),
            preferred_element_type=jnp.float32,
        )[:, 0, :]
        o_ref[...] = red + acc_ref[0, :, :OD] + acc_ref[1, :, :OD]

    odp = acc.shape[-1]
    return pl.pallas_call(
        body,
        grid=(B // block_b,),
        in_specs=[
            pl.BlockSpec((1, IN), lambda i: (0, 0)),
            pl.BlockSpec((block_b, OD, IN), lambda i: (i, 0, 0)),
            pl.BlockSpec((NC, block_b, odp), lambda i: (0, i, 0)),
        ],
        out_specs=pl.BlockSpec((block_b, OD), lambda i: (i, 0)),
        out_shape=jax.ShapeDtypeStruct((B, OD), jnp.float32),
    )(weight, X, acc)


def kernel(X, index, mask, weight):
    B, OD = mask.shape
    odp = 128
    idx32 = index.astype(jnp.int32)
    mask_p = jnp.pad(mask, ((0, 0), (0, odp - OD)))
    zeros = jnp.zeros((B, odp), jnp.float32)
    acc = _sc_scatter_acc(idx32, mask_p, zeros)
    return _tc_reduce_add(X, weight, acc, block_b=256)


# speedup vs baseline: 1.4172x; 1.4172x over previous
import functools

import jax
import jax.numpy as jnp
from jax import lax
from jax.experimental import pallas as pl
from jax.experimental.pallas import tpu as pltpu
from jax.experimental.pallas import tpu_sc as plsc

NC = 2
NS = 16


def _sc_scatter_acc(idx, mask, zeros):
    B, OD = mask.shape
    NW = NC * NS
    rpt = B // NW
    rps = B // NS

    mesh = plsc.VectorSubcoreMesh(core_axis_name="c", subcore_axis_name="s")

    @functools.partial(
        pl.kernel,
        out_type=jax.ShapeDtypeStruct((NC, B, OD), jnp.float32),
        mesh=mesh,
        scratch_types=[
            pltpu.VMEM((rpt,), jnp.int32),
            pltpu.VMEM((rpt, OD), jnp.float32),
            pltpu.VMEM_SHARED((B, OD), jnp.float32),
        ],
    )
    def sc_scatter(idx_hbm, mask_hbm, zero_hbm, acc_hbm, idx_v, mask_v, acc_sh):
        c = lax.axis_index("c")
        s = lax.axis_index("s")
        tid = c * NS + s
        pltpu.sync_copy(zero_hbm.at[pl.ds(s * rps, rps)],
                        acc_sh.at[pl.ds(s * rps, rps)])
        pltpu.sync_copy(idx_hbm.at[pl.ds(tid * rpt, rpt)], idx_v)
        pltpu.sync_copy(mask_hbm.at[pl.ds(tid * rpt, rpt)], mask_v)
        plsc.subcore_barrier()
        pltpu.sync_copy(mask_v, acc_sh.at[idx_v], add=True)
        plsc.subcore_barrier()
        pltpu.sync_copy(acc_sh.at[pl.ds(s * rps, rps)],
                        acc_hbm.at[c, pl.ds(s * rps, rps)])

    return sc_scatter(idx, mask, zeros)


def _tc_reduce_add(X2, w2, acc, OD, block_b):
    B, K = X2.shape

    def body(w_ref, x_ref, acc_ref, o_ref):
        red = jax.lax.dot_general(
            x_ref[...], w_ref[...],
            dimension_numbers=(((1,), (0,)), ((), ())),
            preferred_element_type=jnp.float32,
        )
        o_ref[...] = red + acc_ref[0, :, :OD] + acc_ref[1, :, :OD]

    odp = acc.shape[-1]
    return pl.pallas_call(
        body,
        grid=(B // block_b,),
        in_specs=[
            pl.BlockSpec((K, OD), lambda i: (0, 0)),
            pl.BlockSpec((block_b, K), lambda i: (i, 0)),
            pl.BlockSpec((NC, block_b, odp), lambda i: (0, i, 0)),
        ],
        out_specs=pl.BlockSpec((block_b, OD), lambda i: (i, 0)),
        out_shape=jax.ShapeDtypeStruct((B, OD), jnp.float32),
    )(w2, X2, acc)


def kernel(X, index, mask, weight):
    B, OD = mask.shape
    IN = X.shape[2]
    odp = 128
    idx32 = index.astype(jnp.int32)
    mask_p = jnp.pad(mask, ((0, 0), (0, odp - OD)))
    zeros = jnp.zeros((B, odp), jnp.float32)
    acc = _sc_scatter_acc(idx32, mask_p, zeros)
    X2 = X.reshape(B, OD * IN)
    w2 = jnp.kron(jnp.eye(OD, dtype=jnp.float32), weight.reshape(IN, 1))
    return _tc_reduce_add(X2, w2, acc, OD, block_b=256)


# speedup vs baseline: 3.6500x; 2.5755x over previous
import functools

import jax
import jax.numpy as jnp
from jax import lax
from jax.experimental import pallas as pl
from jax.experimental.pallas import tpu as pltpu
from jax.experimental.pallas import tpu_sc as plsc

NC = 2
NS = 16


def _sc_scatter_acc(idx, mask, zeros):
    B, ODP = mask.shape
    NW = NC * NS
    rpt = B // NW
    rps = B // NS

    mesh = plsc.VectorSubcoreMesh(core_axis_name="c", subcore_axis_name="s")

    @functools.partial(
        pl.kernel,
        out_type=jax.ShapeDtypeStruct((NC, B, ODP), jnp.float32),
        mesh=mesh,
        scratch_types=[
            pltpu.VMEM((rpt,), jnp.int32),
            pltpu.VMEM((rpt, ODP), jnp.float32),
            pltpu.VMEM_SHARED((B, ODP), jnp.float32),
        ],
    )
    def sc_scatter(idx_hbm, mask_hbm, zero_hbm, acc_hbm, idx_v, mask_v, acc_sh):
        c = lax.axis_index("c")
        s = lax.axis_index("s")
        tid = c * NS + s
        pltpu.sync_copy(zero_hbm.at[pl.ds(s * rps, rps)],
                        acc_sh.at[pl.ds(s * rps, rps)])
        pltpu.sync_copy(idx_hbm.at[pl.ds(tid * rpt, rpt)], idx_v)
        pltpu.sync_copy(mask_hbm.at[pl.ds(tid * rpt, rpt)], mask_v)
        plsc.subcore_barrier()
        pltpu.sync_copy(mask_v, acc_sh.at[idx_v], add=True)
        plsc.subcore_barrier()
        pltpu.sync_copy(acc_sh.at[pl.ds(s * rps, rps)],
                        acc_hbm.at[c, pl.ds(s * rps, rps)])

    return sc_scatter(idx, mask, zeros)


def _tc_reduce_t(Xt, wt, block_lanes):
    OD, IN, B = Xt.shape

    def body(w_ref, x_ref, o_ref):
        x = x_ref[...]
        w = w_ref[...]
        o_ref[...] = jnp.sum(x * w[None, :, :], axis=1)

    return pl.pallas_call(
        body,
        grid=(B // block_lanes,),
        in_specs=[
            pl.BlockSpec((IN, 1), lambda i: (0, 0)),
            pl.BlockSpec((OD, IN, block_lanes), lambda i: (0, 0, i)),
        ],
        out_specs=pl.BlockSpec((OD, block_lanes), lambda i: (0, i)),
        out_shape=jax.ShapeDtypeStruct((OD, B), jnp.float32),
    )(wt, Xt)


def _tc_combine_t(red_t, acc_t, block_lanes):
    OD, B = red_t.shape
    odp = acc_t.shape[1]

    def body(r_ref, a_ref, o_ref):
        o_ref[...] = r_ref[...] + a_ref[0, :OD, :] + a_ref[1, :OD, :]

    return pl.pallas_call(
        body,
        grid=(B // block_lanes,),
        in_specs=[
            pl.BlockSpec((OD, block_lanes), lambda i: (0, i)),
            pl.BlockSpec((NC, odp, block_lanes), lambda i: (0, 0, i)),
        ],
        out_specs=pl.BlockSpec((OD, block_lanes), lambda i: (0, i)),
        out_shape=jax.ShapeDtypeStruct((OD, B), jnp.float32),
    )(red_t, acc_t)


def kernel(X, index, mask, weight):
    B, OD = mask.shape
    IN = X.shape[2]
    odp = 128
    idx32 = index.astype(jnp.int32)
    mask_p = jnp.pad(mask, ((0, 0), (0, odp - OD)))
    zeros = jnp.zeros((B, odp), jnp.float32)
    acc = _sc_scatter_acc(idx32, mask_p, zeros)
    Xt = jnp.transpose(X, (1, 2, 0))
    wt = weight.reshape(IN, 1)
    red_t = _tc_reduce_t(Xt, wt, block_lanes=256)
    acc_t = jnp.transpose(acc, (0, 2, 1))
    out_t = _tc_combine_t(red_t, acc_t, block_lanes=2048)
    return jnp.transpose(out_t)


# speedup vs baseline: 3.8564x; 1.0565x over previous
import functools

import jax
import jax.numpy as jnp
from jax import lax
from jax.experimental import pallas as pl
from jax.experimental.pallas import tpu as pltpu
from jax.experimental.pallas import tpu_sc as plsc

NC = 2
NS = 16


def _sc_scatter_acc(idx, mask, zeros):
    B, ODP = mask.shape
    NW = NC * NS
    rpt = B // NW
    rps = B // NS

    mesh = plsc.VectorSubcoreMesh(core_axis_name="c", subcore_axis_name="s")

    @functools.partial(
        pl.kernel,
        out_type=jax.ShapeDtypeStruct((NC, B, ODP), jnp.float32),
        mesh=mesh,
        scratch_types=[
            pltpu.VMEM((rpt,), jnp.int32),
            pltpu.VMEM((rpt, ODP), jnp.float32),
            pltpu.VMEM_SHARED((B, ODP), jnp.float32),
        ],
    )
    def sc_scatter(idx_hbm, mask_hbm, zero_hbm, acc_hbm, idx_v, mask_v, acc_sh):
        c = lax.axis_index("c")
        s = lax.axis_index("s")
        tid = c * NS + s
        pltpu.sync_copy(zero_hbm.at[pl.ds(s * rps, rps)],
                        acc_sh.at[pl.ds(s * rps, rps)])
        pltpu.sync_copy(idx_hbm.at[pl.ds(tid * rpt, rpt)], idx_v)
        pltpu.sync_copy(mask_hbm.at[pl.ds(tid * rpt, rpt)], mask_v)
        plsc.subcore_barrier()
        pltpu.sync_copy(mask_v, acc_sh.at[idx_v], add=True)
        plsc.subcore_barrier()
        pltpu.sync_copy(acc_sh.at[pl.ds(s * rps, rps)],
                        acc_hbm.at[c, pl.ds(s * rps, rps)])

    return sc_scatter(idx, mask, zeros)


def _tc_reduce_t(Xt, wt, block_lanes):
    OD, IN, B = Xt.shape

    def body(w_ref, x_ref, o_ref):
        x = x_ref[...]
        w = w_ref[...]
        o_ref[...] = jnp.sum(x * w[None, :, :], axis=1)

    return pl.pallas_call(
        body,
        grid=(B // block_lanes,),
        in_specs=[
            pl.BlockSpec((IN, 1), lambda i: (0, 0)),
            pl.BlockSpec((OD, IN, block_lanes), lambda i: (0, 0, i)),
        ],
        out_specs=pl.BlockSpec((OD, block_lanes), lambda i: (0, i)),
        out_shape=jax.ShapeDtypeStruct((OD, B), jnp.float32),
        compiler_params=pltpu.CompilerParams(
            vmem_limit_bytes=100 * 1024 * 1024,
        ),
    )(wt, Xt)


def _tc_combine_t(red_t, acc, block_lanes):
    OD, B = red_t.shape
    odp = acc.shape[-1]

    def body(r_ref, a_ref, o_ref):
        a = a_ref[...]
        at = jnp.transpose(a[0] + a[1])
        o_ref[...] = r_ref[...] + at[:OD, :]

    return pl.pallas_call(
        body,
        grid=(B // block_lanes,),
        in_specs=[
            pl.BlockSpec((OD, block_lanes), lambda i: (0, i)),
            pl.BlockSpec((NC, block_lanes, odp), lambda i: (0, i, 0)),
        ],
        out_specs=pl.BlockSpec((OD, block_lanes), lambda i: (0, i)),
        out_shape=jax.ShapeDtypeStruct((OD, B), jnp.float32),
    )(red_t, acc)


def kernel(X, index, mask, weight):
    B, OD = mask.shape
    IN = X.shape[2]
    odp = 128
    idx32 = index.astype(jnp.int32)
    mask_p = jnp.pad(mask, ((0, 0), (0, odp - OD)))
    zeros = jnp.zeros((B, odp), jnp.float32)
    acc = _sc_scatter_acc(idx32, mask_p, zeros)
    Xt = jnp.transpose(X, (1, 2, 0))
    wt = weight.reshape(IN, 1)
    red_t = _tc_reduce_t(Xt, wt, block_lanes=512)
    out_t = _tc_combine_t(red_t, acc, block_lanes=1024)
    return jnp.transpose(out_t)


# speedup vs baseline: 4.0317x; 1.0455x over previous
import functools

import jax
import jax.numpy as jnp
from jax import lax
from jax.experimental import pallas as pl
from jax.experimental.pallas import tpu as pltpu
from jax.experimental.pallas import tpu_sc as plsc

NC = 2
NS = 16


def _sc_scatter_acc(idx, mask):
    B, ODP = mask.shape
    NW = NC * NS
    rpt = B // NW
    rps = B // NS

    mesh = plsc.VectorSubcoreMesh(core_axis_name="c", subcore_axis_name="s")

    @functools.partial(
        pl.kernel,
        out_type=jax.ShapeDtypeStruct((NC, B, ODP), jnp.float32),
        mesh=mesh,
        scratch_types=[
            pltpu.VMEM((rpt,), jnp.int32),
            pltpu.VMEM((rpt, ODP), jnp.float32),
            pltpu.VMEM((rpt, ODP), jnp.float32),
            pltpu.VMEM_SHARED((B, ODP), jnp.float32),
        ],
    )
    def sc_scatter(idx_hbm, mask_hbm, acc_hbm, idx_v, mask_v, zb, acc_sh):
        c = lax.axis_index("c")
        s = lax.axis_index("s")
        tid = c * NS + s
        zeros16 = jnp.zeros((16,), jnp.float32)

        def zero_row(r, _):
            for k in range(ODP // 16):
                zb[r, pl.ds(k * 16, 16)] = zeros16
            return _
        lax.fori_loop(0, rpt, zero_row, None)
        for h in range(rps // rpt):
            pltpu.sync_copy(zb, acc_sh.at[pl.ds(s * rps + h * rpt, rpt)])
        pltpu.sync_copy(idx_hbm.at[pl.ds(tid * rpt, rpt)], idx_v)
        pltpu.sync_copy(mask_hbm.at[pl.ds(tid * rpt, rpt)], mask_v)
        plsc.subcore_barrier()
        pltpu.sync_copy(mask_v, acc_sh.at[idx_v], add=True)
        plsc.subcore_barrier()
        pltpu.sync_copy(acc_sh.at[pl.ds(s * rps, rps)],
                        acc_hbm.at[c, pl.ds(s * rps, rps)])

    return sc_scatter(idx, mask)


def _tc_reduce_t(Xt, wt, block_lanes):
    OD, IN, B = Xt.shape

    def body(w_ref, x_ref, o_ref):
        x = x_ref[...]
        w = w_ref[...]
        o_ref[...] = jnp.sum(x * w[None, :, :], axis=1)

    return pl.pallas_call(
        body,
        grid=(B // block_lanes,),
        in_specs=[
            pl.BlockSpec((IN, 1), lambda i: (0, 0)),
            pl.BlockSpec((OD, IN, block_lanes), lambda i: (0, 0, i)),
        ],
        out_specs=pl.BlockSpec((OD, block_lanes), lambda i: (0, i)),
        out_shape=jax.ShapeDtypeStruct((OD, B), jnp.float32),
        compiler_params=pltpu.CompilerParams(
            vmem_limit_bytes=100 * 1024 * 1024,
        ),
    )(wt, Xt)


def _tc_combine_t(red_t, acc, block_lanes):
    OD, B = red_t.shape
    odp = acc.shape[-1]

    def body(r_ref, a_ref, o_ref):
        a = a_ref[...]
        at = jnp.transpose(a[0] + a[1])
        o_ref[...] = r_ref[...] + at[:OD, :]

    return pl.pallas_call(
        body,
        grid=(B // block_lanes,),
        in_specs=[
            pl.BlockSpec((OD, block_lanes), lambda i: (0, i)),
            pl.BlockSpec((NC, block_lanes, odp), lambda i: (0, i, 0)),
        ],
        out_specs=pl.BlockSpec((OD, block_lanes), lambda i: (0, i)),
        out_shape=jax.ShapeDtypeStruct((OD, B), jnp.float32),
    )(red_t, acc)


def kernel(X, index, mask, weight):
    B, OD = mask.shape
    IN = X.shape[2]
    odp = 128
    idx32 = index.astype(jnp.int32)
    mask_p = jnp.pad(mask, ((0, 0), (0, odp - OD)))
    acc = _sc_scatter_acc(idx32, mask_p)
    Xt = jnp.transpose(X, (1, 2, 0))
    wt = weight.reshape(IN, 1)
    red_t = _tc_reduce_t(Xt, wt, block_lanes=512)
    out_t = _tc_combine_t(red_t, acc, block_lanes=1024)
    return jnp.transpose(out_t)
